# baseline (speedup 1.0000x reference)
import functools

import jax
import jax.numpy as jnp
from jax import lax
from jax.experimental import pallas as pl
from jax.experimental.pallas import tpu as pltpu
from jax.experimental.pallas import tpu_sc as plsc

TB = 512


def _sc_gather(flat_table, idx_raw, num_cat, vocab):
    R = idx_raw.shape[0]
    D = flat_table.shape[1]
    NW = 32
    b_per_w = R // NW
    CH = 128
    n_ch = b_per_w // CH
    mesh = plsc.VectorSubcoreMesh(core_axis_name="c", subcore_axis_name="s")

    @functools.partial(
        pl.kernel, mesh=mesh,
        compiler_params=pltpu.CompilerParams(use_tc_tiling_on_sc=False),
        out_type=jax.ShapeDtypeStruct((R, D), jnp.float32),
        scratch_types=[
            pltpu.VMEM((b_per_w,), jnp.int32),
            pltpu.VMEM((CH, D), jnp.float32),
            pltpu.VMEM((CH, D), jnp.float32),
            pltpu.SemaphoreType.DMA,
        ],
    )
    def gk(idx_hbm, tab_hbm, out_hbm, idx_v, rows0, rows1, sem):
        wid = lax.axis_index("s") * 2 + lax.axis_index("c")
        base = wid * b_per_w
        pltpu.sync_copy(idx_hbm.at[pl.ds(base, b_per_w)], idx_v)
        offv = lax.rem(lax.iota(jnp.int32, 16), num_cat) * vocab

        def add_off(i, carry):
            s = pl.multiple_of(i * 16, 16)
            idx_v[pl.ds(s, 16)] = idx_v[pl.ds(s, 16)] + offv
            return carry

        lax.fori_loop(0, b_per_w // 16, add_off, 0)

        bufs = (rows0, rows1)
        handles = [None, None]
        handles[0] = pltpu.async_copy(tab_hbm.at[idx_v.at[pl.ds(0, CH)]], bufs[0], sem)
        for s in range(n_ch):
            nxt = s + 1
            if nxt < n_ch:
                handles[nxt % 2] = pltpu.async_copy(
                    tab_hbm.at[idx_v.at[pl.ds(nxt * CH, CH)]], bufs[nxt % 2], sem)
            handles[s % 2].wait()
            pltpu.sync_copy(bufs[s % 2], out_hbm.at[pl.ds(base + s * CH, CH)])

    return gk(idx_raw, flat_table)


def _mlp_body(comb, wea, ven,
              wW1, wb1, wW2, wb2, vW1, vb1, vW2, vb2,
              W0abc, W0d, W0e, b0, W1, b1, W2, b2, out):
    f32, bf16 = jnp.float32, jnp.bfloat16
    dot = functools.partial(jnp.dot, preferred_element_type=f32)
    we = jnp.maximum(dot(wea[...], wW1[...]) + wb1[...], 0.0)
    we = dot(we, wW2[...]) + wb2[...]
    ve = jnp.maximum(dot(ven[...], vW1[...]) + vb1[...], 0.0)
    ve = dot(ve, vW2[...]) + vb2[...]
    h0 = (dot(comb[...], W0abc[...])
          + dot(we.astype(bf16), W0d[...])
          + dot(ve.astype(bf16), W0e[...])
          + b0[...])
    h0 = jnp.maximum(h0, 0.0).astype(bf16)
    h1 = jnp.maximum(dot(h0, W1[...]) + b1[...], 0.0).astype(bf16)
    out[...] = dot(h1, W2[...]) + b2[...]


def _mlp(comb, wea, ven,
         wW1, wb1, wW2, wb2, vW1, vb1, vW2, vb2,
         W0abc, W0d, W0e, b0, W1, b1, W2, b2):
    B = comb.shape[0]
    CTX = W2.shape[1]

    def bspec(a):
        return pl.BlockSpec((TB,) + a.shape[1:], lambda i: (i,) + (0,) * (a.ndim - 1))

    def fspec(a):
        return pl.BlockSpec(a.shape, lambda i: (0,) * a.ndim)

    batched = [comb, wea, ven]
    full = [wW1, wb1, wW2, wb2, vW1, vb1, vW2, vb2,
            W0abc, W0d, W0e, b0, W1, b1, W2, b2]
    return pl.pallas_call(
        _mlp_body,
        grid=(B // TB,),
        in_specs=[bspec(a) for a in batched] + [fspec(a) for a in full],
        out_specs=pl.BlockSpec((TB, CTX), lambda i: (i, 0)),
        out_shape=jax.ShapeDtypeStruct((B, CTX), jnp.float32),
    )(*batched, *full)


def kernel(numeric_features, categorical_features, video_features, video_mask,
           weather_features, venue_coordinates, emb_tables,
           wW1, wb1, wW2, wb2, vW1, vb1, vW2, vb2,
           mW0, mb0, mW1, mb1, mW2, mb2):
    B = numeric_features.shape[0]
    num_cat, vocab, emb_d = emb_tables.shape
    numeric = numeric_features.shape[1]
    video = video_features.shape[1]
    weather = weather_features.shape[1]
    venue = venue_coordinates.shape[1]

    flat_tab = emb_tables.reshape(num_cat * vocab, emb_d)
    idx_raw = categorical_features.astype(jnp.int32).reshape(-1)
    emb = _sc_gather(flat_tab, idx_raw, num_cat, vocab).reshape(B, num_cat * emb_d)

    comb = jnp.concatenate(
        [numeric_features, emb, video_features * video_mask], axis=1
    ).astype(jnp.bfloat16)

    o2 = numeric + num_cat * emb_d + video
    o3 = o2 + weather
    o4 = o3 + 2 * venue
    mW0h = mW0.astype(jnp.bfloat16)
    W0abc, W0d, W0e = mW0h[:o2], mW0h[o2:o3], mW0h[o3:o4]
    mW1 = mW1.astype(jnp.bfloat16)
    mW2 = mW2.astype(jnp.bfloat16)

    r1 = lambda b: b.reshape(1, -1)
    return _mlp(comb, weather_features, venue_coordinates,
                wW1, r1(wb1), wW2, r1(wb2), vW1, r1(vb1), vW2, r1(vb2),
                W0abc, W0d, W0e, r1(mb0), mW1, r1(mb1), mW2, r1(mb2))


# speedup vs baseline: 1.1476x; 1.1476x over previous
import functools

import jax
import jax.numpy as jnp
from jax import lax
from jax.experimental import pallas as pl
from jax.experimental.pallas import tpu as pltpu
from jax.experimental.pallas import tpu_sc as plsc

TB = 1024


def _sc_gather(flat_table, idx_raw, num_cat, vocab):
    R = idx_raw.shape[0]
    D = flat_table.shape[1]
    NW = 32
    b_per_w = R // NW
    CH = 128
    n_ch = b_per_w // CH
    mesh = plsc.VectorSubcoreMesh(core_axis_name="c", subcore_axis_name="s")

    @functools.partial(
        pl.kernel, mesh=mesh,
        compiler_params=pltpu.CompilerParams(use_tc_tiling_on_sc=False),
        out_type=jax.ShapeDtypeStruct((R, D), jnp.float32),
        scratch_types=[
            pltpu.VMEM((b_per_w,), jnp.int32),
            pltpu.VMEM((CH, D), jnp.float32),
            pltpu.VMEM((CH, D), jnp.float32),
            pltpu.SemaphoreType.DMA,
        ],
    )
    def gk(idx_hbm, tab_hbm, out_hbm, idx_v, rows0, rows1, sem):
        wid = lax.axis_index("s") * 2 + lax.axis_index("c")
        base = wid * b_per_w
        pltpu.sync_copy(idx_hbm.at[pl.ds(base, b_per_w)], idx_v)
        offv = lax.rem(lax.iota(jnp.int32, 16), num_cat) * vocab

        def add_off(i, carry):
            s = pl.multiple_of(i * 16, 16)
            idx_v[pl.ds(s, 16)] = idx_v[pl.ds(s, 16)] + offv
            return carry

        lax.fori_loop(0, b_per_w // 16, add_off, 0)

        bufs = (rows0, rows1)
        handles = [None, None]
        handles[0] = pltpu.async_copy(tab_hbm.at[idx_v.at[pl.ds(0, CH)]], bufs[0], sem)
        for s in range(n_ch):
            nxt = s + 1
            if nxt < n_ch:
                handles[nxt % 2] = pltpu.async_copy(
                    tab_hbm.at[idx_v.at[pl.ds(nxt * CH, CH)]], bufs[nxt % 2], sem)
            handles[s % 2].wait()
            pltpu.sync_copy(bufs[s % 2], out_hbm.at[pl.ds(base + s * CH, CH)])

    return gk(idx_raw, flat_table)


def _mlp_body(nm, emb, vid, msk, wea, ven,
              wW1, wb1, wW2, wb2, vW1, vb1, vW2, vb2,
              W0a, W0b, W0c, W0d, W0e, b0, W1, b1, W2, b2, out):
    f32, bf16 = jnp.float32, jnp.bfloat16
    dot = functools.partial(jnp.dot, preferred_element_type=f32)
    we = jnp.maximum(dot(wea[...], wW1[...]) + wb1[...], 0.0)
    we = dot(we, wW2[...]) + wb2[...]
    ve = jnp.maximum(dot(ven[...], vW1[...]) + vb1[...], 0.0)
    ve = dot(ve, vW2[...]) + vb2[...]
    h0 = (dot(nm[...].astype(bf16), W0a[...])
          + dot(emb[...].astype(bf16), W0b[...])
          + dot((vid[...] * msk[...]).astype(bf16), W0c[...])
          + dot(we.astype(bf16), W0d[...])
          + dot(ve.astype(bf16), W0e[...])
          + b0[...])
    h0 = jnp.maximum(h0, 0.0).astype(bf16)
    h1 = jnp.maximum(dot(h0, W1[...]) + b1[...], 0.0).astype(bf16)
    out[...] = dot(h1, W2[...]) + b2[...]


def _mlp(nm, emb, vid, msk, wea, ven,
         wW1, wb1, wW2, wb2, vW1, vb1, vW2, vb2,
         W0a, W0b, W0c, W0d, W0e, b0, W1, b1, W2, b2):
    B = nm.shape[0]
    CTX = W2.shape[1]

    def bspec(a):
        return pl.BlockSpec((TB,) + a.shape[1:], lambda i: (i,) + (0,) * (a.ndim - 1))

    def fspec(a):
        return pl.BlockSpec(a.shape, lambda i: (0,) * a.ndim)

    batched = [nm, emb, vid, msk, wea, ven]
    full = [wW1, wb1, wW2, wb2, vW1, vb1, vW2, vb2,
            W0a, W0b, W0c, W0d, W0e, b0, W1, b1, W2, b2]
    return pl.pallas_call(
        _mlp_body,
        grid=(B // TB,),
        in_specs=[bspec(a) for a in batched] + [fspec(a) for a in full],
        out_specs=pl.BlockSpec((TB, CTX), lambda i: (i, 0)),
        out_shape=jax.ShapeDtypeStruct((B, CTX), jnp.float32),
    )(*batched, *full)


def kernel(numeric_features, categorical_features, video_features, video_mask,
           weather_features, venue_coordinates, emb_tables,
           wW1, wb1, wW2, wb2, vW1, vb1, vW2, vb2,
           mW0, mb0, mW1, mb1, mW2, mb2):
    B = numeric_features.shape[0]
    num_cat, vocab, emb_d = emb_tables.shape
    numeric = numeric_features.shape[1]
    video = video_features.shape[1]
    weather = weather_features.shape[1]
    venue = venue_coordinates.shape[1]

    flat_tab = emb_tables.reshape(num_cat * vocab, emb_d)
    idx_raw = categorical_features.astype(jnp.int32).reshape(-1)
    emb = _sc_gather(flat_tab, idx_raw, num_cat, vocab).reshape(B, num_cat * emb_d)

    o0 = numeric
    o1 = o0 + num_cat * emb_d
    o2 = o1 + video
    o3 = o2 + weather
    o4 = o3 + 2 * venue
    mW0h = mW0.astype(jnp.bfloat16)
    W0a, W0b, W0c = mW0h[:o0], mW0h[o0:o1], mW0h[o1:o2]
    W0d, W0e = mW0h[o2:o3], mW0h[o3:o4]
    mW1 = mW1.astype(jnp.bfloat16)
    mW2 = mW2.astype(jnp.bfloat16)

    r1 = lambda b: b.reshape(1, -1)
    return _mlp(numeric_features, emb, video_features, video_mask,
                weather_features, venue_coordinates,
                wW1, r1(wb1), wW2, r1(wb2), vW1, r1(vb1), vW2, r1(vb2),
                W0a, W0b, W0c, W0d, W0e, r1(mb0), mW1, r1(mb1), mW2, r1(mb2))


# speedup vs baseline: 1.1508x; 1.0028x over previous
import functools

import jax
import jax.numpy as jnp
from jax import lax
from jax.experimental import pallas as pl
from jax.experimental.pallas import tpu as pltpu
from jax.experimental.pallas import tpu_sc as plsc

TB = 1024


def _sc_gather(flat_table, idx_raw, num_cat, vocab):
    R = idx_raw.shape[0]
    D = flat_table.shape[1]
    NW = 32
    b_per_w = R // NW
    CH = 128
    n_ch = b_per_w // CH
    mesh = plsc.VectorSubcoreMesh(core_axis_name="c", subcore_axis_name="s")

    @functools.partial(
        pl.kernel, mesh=mesh,
        compiler_params=pltpu.CompilerParams(use_tc_tiling_on_sc=False),
        out_type=jax.ShapeDtypeStruct((R, D), jnp.float32),
        scratch_types=[
            pltpu.VMEM((b_per_w,), jnp.int32),
            pltpu.VMEM((CH, D), jnp.float32),
            pltpu.VMEM((CH, D), jnp.float32),
            pltpu.SemaphoreType.DMA,
        ],
    )
    def gk(idx_hbm, tab_hbm, out_hbm, idx_v, rows0, rows1, sem):
        wid = lax.axis_index("s") * 2 + lax.axis_index("c")
        base = wid * b_per_w
        pltpu.sync_copy(idx_hbm.at[pl.ds(base, b_per_w)], idx_v)
        offv = lax.rem(lax.iota(jnp.int32, 16), num_cat) * vocab

        def add_off(i, carry):
            s = pl.multiple_of(i * 16, 16)
            idx_v[pl.ds(s, 16)] = idx_v[pl.ds(s, 16)] + offv
            return carry

        lax.fori_loop(0, b_per_w // 16, add_off, 0)

        bufs = (rows0, rows1)
        handles = [None, None]
        handles[0] = pltpu.async_copy(tab_hbm.at[idx_v.at[pl.ds(0, CH)]], bufs[0], sem)
        for s in range(n_ch):
            nxt = s + 1
            if nxt < n_ch:
                handles[nxt % 2] = pltpu.async_copy(
                    tab_hbm.at[idx_v.at[pl.ds(nxt * CH, CH)]], bufs[nxt % 2], sem)
            handles[s % 2].wait()
            pltpu.sync_copy(bufs[s % 2], out_hbm.at[pl.ds(base + s * CH, CH)])

    return gk(idx_raw, flat_table)


def _mlp_body(nm, emb, vid, msk, wea, ven,
              wW1, wb1, wW2, wb2, vW1, vb1, vW2, vb2,
              W0abc, W0d, W0e, b0, W1, b1, W2, b2, out):
    f32, bf16 = jnp.float32, jnp.bfloat16
    dot = functools.partial(jnp.dot, preferred_element_type=f32)
    we = jnp.maximum(dot(wea[...], wW1[...]) + wb1[...], 0.0)
    we = dot(we, wW2[...]) + wb2[...]
    ve = jnp.maximum(dot(ven[...], vW1[...]) + vb1[...], 0.0)
    ve = dot(ve, vW2[...]) + vb2[...]
    xs = jnp.concatenate(
        [nm[...].astype(bf16), emb[...].astype(bf16),
         (vid[...] * msk[...]).astype(bf16)], axis=1)
    h0 = (dot(xs, W0abc[...])
          + dot(we.astype(bf16), W0d[...])
          + dot(ve.astype(bf16), W0e[...])
          + b0[...])
    h0 = jnp.maximum(h0, 0.0).astype(bf16)
    h1 = jnp.maximum(dot(h0, W1[...]) + b1[...], 0.0).astype(bf16)
    out[...] = dot(h1, W2[...]) + b2[...]


def _mlp(nm, emb, vid, msk, wea, ven,
         wW1, wb1, wW2, wb2, vW1, vb1, vW2, vb2,
         W0abc, W0d, W0e, b0, W1, b1, W2, b2):
    B = nm.shape[0]
    CTX = W2.shape[1]

    def bspec(a):
        return pl.BlockSpec((TB,) + a.shape[1:], lambda i: (i,) + (0,) * (a.ndim - 1))

    def fspec(a):
        return pl.BlockSpec(a.shape, lambda i: (0,) * a.ndim)

    batched = [nm, emb, vid, msk, wea, ven]
    full = [wW1, wb1, wW2, wb2, vW1, vb1, vW2, vb2,
            W0abc, W0d, W0e, b0, W1, b1, W2, b2]
    return pl.pallas_call(
        _mlp_body,
        grid=(B // TB,),
        in_specs=[bspec(a) for a in batched] + [fspec(a) for a in full],
        out_specs=pl.BlockSpec((TB, CTX), lambda i: (i, 0)),
        out_shape=jax.ShapeDtypeStruct((B, CTX), jnp.float32),
    )(*batched, *full)


def kernel(numeric_features, categorical_features, video_features, video_mask,
           weather_features, venue_coordinates, emb_tables,
           wW1, wb1, wW2, wb2, vW1, vb1, vW2, vb2,
           mW0, mb0, mW1, mb1, mW2, mb2):
    B = numeric_features.shape[0]
    num_cat, vocab, emb_d = emb_tables.shape
    numeric = numeric_features.shape[1]
    video = video_features.shape[1]
    weather = weather_features.shape[1]
    venue = venue_coordinates.shape[1]

    flat_tab = emb_tables.reshape(num_cat * vocab, emb_d)
    idx_raw = categorical_features.astype(jnp.int32).reshape(-1)
    emb = _sc_gather(flat_tab, idx_raw, num_cat, vocab).reshape(B, num_cat * emb_d)

    o0 = numeric
    o1 = o0 + num_cat * emb_d
    o2 = o1 + video
    o3 = o2 + weather
    o4 = o3 + 2 * venue
    mW0h = mW0.astype(jnp.bfloat16)
    W0abc, W0d, W0e = mW0h[:o2], mW0h[o2:o3], mW0h[o3:o4]
    mW1 = mW1.astype(jnp.bfloat16)
    mW2 = mW2.astype(jnp.bfloat16)

    r1 = lambda b: b.reshape(1, -1)
    return _mlp(numeric_features, emb, video_features, video_mask,
                weather_features, venue_coordinates,
                wW1, r1(wb1), wW2, r1(wb2), vW1, r1(vb1), vW2, r1(vb2),
                W0abc, W0d, W0e, r1(mb0), mW1, r1(mb1), mW2, r1(mb2))


# speedup vs baseline: 1.1520x; 1.0011x over previous
import functools

import jax
import jax.numpy as jnp
from jax import lax
from jax.experimental import pallas as pl
from jax.experimental.pallas import tpu as pltpu
from jax.experimental.pallas import tpu_sc as plsc

TB = 1024


def _sc_gather(flat_table, idx_raw, num_cat, vocab):
    R = idx_raw.shape[0]
    D = flat_table.shape[1]
    NW = 32
    b_per_w = R // NW
    CH = 128
    n_ch = b_per_w // CH
    mesh = plsc.VectorSubcoreMesh(core_axis_name="c", subcore_axis_name="s")

    @functools.partial(
        pl.kernel, mesh=mesh,
        compiler_params=pltpu.CompilerParams(use_tc_tiling_on_sc=False),
        out_type=jax.ShapeDtypeStruct((R, D), jnp.float32),
    scratch_types=[
            pltpu.VMEM((b_per_w,), jnp.int32),
            pltpu.VMEM((CH, D), jnp.float32),
            pltpu.VMEM((CH, D), jnp.float32),
            pltpu.VMEM((CH, D), jnp.float32),
            pltpu.VMEM((CH, D), jnp.float32),
            pltpu.SemaphoreType.DMA,
            pltpu.SemaphoreType.DMA,
        ],
    )
    def gk(idx_hbm, tab_hbm, out_hbm, idx_v, rows0, rows1, rows2, rows3,
           gsem, ssem):
        wid = lax.axis_index("s") * 2 + lax.axis_index("c")
        base = wid * b_per_w
        pltpu.sync_copy(idx_hbm.at[pl.ds(base, b_per_w)], idx_v)
        offv = lax.rem(lax.iota(jnp.int32, 16), num_cat) * vocab

        def add_off(i, carry):
            s = pl.multiple_of(i * 16, 16)
            idx_v[pl.ds(s, 16)] = idx_v[pl.ds(s, 16)] + offv
            return carry

        lax.fori_loop(0, b_per_w // 16, add_off, 0)

        bufs = (rows0, rows1, rows2, rows3)

        def gather(s):
            return pltpu.async_copy(
                tab_hbm.at[idx_v.at[pl.ds(s * CH, CH)]], bufs[s % 4], gsem)

        hg, hs = {}, {}
        for s in range(min(2, n_ch)):
            hg[s] = gather(s)
        for s in range(n_ch):
            nxt = s + 2
            if nxt < n_ch:
                if nxt - 4 >= 0:
                    hs.pop(nxt - 4).wait()
                hg[nxt] = gather(nxt)
            hg.pop(s).wait()
            hs[s] = pltpu.async_copy(
                bufs[s % 4], out_hbm.at[pl.ds(base + s * CH, CH)], ssem)
        for s in sorted(hs):
            hs[s].wait()

    return gk(idx_raw, flat_table)


def _mlp_body(nm, emb, vid, msk, wea, ven,
              wW1, wb1, wW2, wb2, vW1, vb1, vW2, vb2,
              W0abc, W0d, W0e, b0, W1, b1, W2, b2, out):
    f32, bf16 = jnp.float32, jnp.bfloat16
    dot = functools.partial(jnp.dot, preferred_element_type=f32)
    we = jnp.maximum(dot(wea[...], wW1[...]) + wb1[...], 0.0)
    we = dot(we, wW2[...]) + wb2[...]
    ve = jnp.maximum(dot(ven[...], vW1[...]) + vb1[...], 0.0)
    ve = dot(ve, vW2[...]) + vb2[...]
    xs = jnp.concatenate(
        [nm[...].astype(bf16), emb[...].astype(bf16),
         (vid[...] * msk[...]).astype(bf16)], axis=1)
    h0 = (dot(xs, W0abc[...])
          + dot(we.astype(bf16), W0d[...])
          + dot(ve.astype(bf16), W0e[...])
          + b0[...])
    h0 = jnp.maximum(h0, 0.0).astype(bf16)
    h1 = jnp.maximum(dot(h0, W1[...]) + b1[...], 0.0).astype(bf16)
    out[...] = dot(h1, W2[...]) + b2[...]


def _mlp(nm, emb, vid, msk, wea, ven,
         wW1, wb1, wW2, wb2, vW1, vb1, vW2, vb2,
         W0abc, W0d, W0e, b0, W1, b1, W2, b2):
    B = nm.shape[0]
    CTX = W2.shape[1]

    def bspec(a):
        return pl.BlockSpec((TB,) + a.shape[1:], lambda i: (i,) + (0,) * (a.ndim - 1))

    def fspec(a):
        return pl.BlockSpec(a.shape, lambda i: (0,) * a.ndim)

    batched = [nm, emb, vid, msk, wea, ven]
    full = [wW1, wb1, wW2, wb2, vW1, vb1, vW2, vb2,
            W0abc, W0d, W0e, b0, W1, b1, W2, b2]
    return pl.pallas_call(
        _mlp_body,
        grid=(B // TB,),
        in_specs=[bspec(a) for a in batched] + [fspec(a) for a in full],
        out_specs=pl.BlockSpec((TB, CTX), lambda i: (i, 0)),
        out_shape=jax.ShapeDtypeStruct((B, CTX), jnp.float32),
    )(*batched, *full)


def kernel(numeric_features, categorical_features, video_features, video_mask,
           weather_features, venue_coordinates, emb_tables,
           wW1, wb1, wW2, wb2, vW1, vb1, vW2, vb2,
           mW0, mb0, mW1, mb1, mW2, mb2):
    B = numeric_features.shape[0]
    num_cat, vocab, emb_d = emb_tables.shape
    numeric = numeric_features.shape[1]
    video = video_features.shape[1]
    weather = weather_features.shape[1]
    venue = venue_coordinates.shape[1]

    flat_tab = emb_tables.reshape(num_cat * vocab, emb_d)
    idx_raw = categorical_features.astype(jnp.int32).reshape(-1)
    emb = _sc_gather(flat_tab, idx_raw, num_cat, vocab).reshape(B, num_cat * emb_d)

    o0 = numeric
    o1 = o0 + num_cat * emb_d
    o2 = o1 + video
    o3 = o2 + weather
    o4 = o3 + 2 * venue
    mW0h = mW0.astype(jnp.bfloat16)
    W0abc, W0d, W0e = mW0h[:o2], mW0h[o2:o3], mW0h[o3:o4]
    mW1 = mW1.astype(jnp.bfloat16)
    mW2 = mW2.astype(jnp.bfloat16)

    r1 = lambda b: b.reshape(1, -1)
    return _mlp(numeric_features, emb, video_features, video_mask,
                weather_features, venue_coordinates,
                wW1, r1(wb1), wW2, r1(wb2), vW1, r1(vb1), vW2, r1(vb2),
                W0abc, W0d, W0e, r1(mb0), mW1, r1(mb1), mW2, r1(mb2))
